# baseline (speedup 1.0000x reference)
import functools
import math

import jax
import jax.numpy as jnp
from jax import lax
from jax.experimental import pallas as pl
from jax.experimental.pallas import tpu as pltpu
from jax.experimental.pallas import tpu_sc as plsc

V = 100000
B = 2048
SV = 0.1 / (V - 2)
CONF = 1.0 - 0.1
C1 = SV * math.log(SV) * (V - 2) + CONF * math.log(CONF)

NW = 32
PER_W = B // NW
LAST_TILE = (V // 128) * 128

R_SC = 1024
ROWS_PW = R_SC // NW
BANDS_PW = ROWS_PW // 8
CTILES = 11
CW = CTILES * 128
NCH = LAST_TILE // CW

BR = 32
GR = (B - R_SC) // BR
SC_BLOCKS = R_SC // BR


@functools.lru_cache(maxsize=1)
def _build_sc_part():
    mesh = plsc.VectorSubcoreMesh(core_axis_name="c", subcore_axis_name="s")

    @functools.partial(
        pl.kernel,
        out_type=[
            jax.ShapeDtypeStruct((NW, 16), jnp.float32),
            jax.ShapeDtypeStruct((R_SC, 16), jnp.float32),
        ],
        mesh=mesh,
        scratch_types=[
            pltpu.VMEM((PER_W,), jnp.int32),
            pltpu.VMEM((PER_W, 8, 128), jnp.float32),
            pltpu.VMEM((16,), jnp.float32),
            pltpu.VMEM((2, 8, CW), jnp.float32),
            pltpu.VMEM((8, 16), jnp.float32),
            pltpu.SemaphoreType.DMA,
            pltpu.SemaphoreType.DMA,
        ],
    )
    def _sc_part(x_hbm, tgt_hbm, gout_hbm, rout_hbm,
                 tgt_v, tile_v, acc_v, buf_v, stage_v, sem_g, sem_r):
        wid = lax.axis_index("s") * 2 + lax.axis_index("c")
        lane = lax.broadcasted_iota(jnp.int32, (16,), 0)

        gbase = wid * PER_W
        pltpu.sync_copy(tgt_hbm.at[pl.ds(gbase, PER_W)], tgt_v)
        copies = []
        for j in range(PER_W):
            t = tgt_v[pl.ds((j // 16) * 16, 16)][j % 16]
            col = pl.multiple_of(
                jnp.minimum((t >> 7) << 7, LAST_TILE - 128), 128)
            row = pl.multiple_of((gbase + j) & ~7, 8)
            copies.append(pltpu.async_copy(
                x_hbm.at[pl.ds(row, 8), pl.ds(col, 128)],
                tile_v.at[j], sem_g))
        for c in copies:
            c.wait()
        acc = jnp.zeros((16,), jnp.float32)
        for j in range(PER_W):
            t = tgt_v[pl.ds((j // 16) * 16, 16)][j % 16]
            cs = pl.multiple_of(((t >> 4) & 7) * 16, 16)
            chunk = tile_v[j, j % 8, pl.ds(cs, 16)]
            ok = (t != 0) & (t < LAST_TILE)
            tt = jnp.where(ok, t & 15, -1)
            acc = acc + jnp.where(lane == tt, chunk, 0.0)
        acc_v[...] = acc
        pltpu.sync_copy(acc_v, gout_hbm.at[wid])

        rbase = wid * ROWS_PW

        def chunk_sum(b, accs, is0f):
            new = []
            for r in range(8):
                s = [buf_v[b, r, pl.ds(k * 16, 16)] for k in range(4)]
                for k in range(4, CW // 16):
                    s[k % 4] = s[k % 4] + buf_v[b, r, pl.ds(k * 16, 16)]
                a = accs[r] + ((s[0] + s[1]) + (s[2] + s[3]))
                a = a - is0f * jnp.where(
                    lane == 0, buf_v[b, r, pl.ds(0, 16)], 0.0)
                new.append(a)
            return tuple(new)

        def band_body(band, carry):
            row0 = pl.multiple_of(rbase + band * 8, 8)

            def issue(c, b):
                ccol = pl.multiple_of(c * CW, 128)
                pltpu.async_copy(
                    x_hbm.at[pl.ds(row0, 8), pl.ds(ccol, CW)],
                    buf_v.at[b], sem_r)

            def wait(b):
                pltpu.make_async_copy(
                    x_hbm.at[pl.ds(0, 8), pl.ds(0, CW)],
                    buf_v.at[b], sem_r).wait()

            issue(0, 0)

            def pair_body(i, accs):
                c = i * 2
                wait(0)
                issue(c + 1, 1)
                accs = chunk_sum(0, accs, jnp.where(i == 0, 1.0, 0.0))
                wait(1)
                issue(c + 2, 0)
                accs = chunk_sum(1, accs, 0.0)
                return accs

            zero = jnp.zeros((16,), jnp.float32)
            accs = lax.fori_loop(0, NCH // 2, pair_body, (zero,) * 8)
            wait(0)
            accs = chunk_sum(0, accs, 0.0)
            for r in range(8):
                stage_v[r, :] = accs[r]
            pltpu.sync_copy(stage_v, rout_hbm.at[pl.ds(row0, 8)])
            return carry

        lax.fori_loop(0, BANDS_PW, band_body, 0)

    return _sc_part


def _tc_body(x_ref, tgt_ref, tail_ref, part_ref, acc_ref):
    pid = pl.program_id(0)

    @pl.when(pid == 0)
    def _():
        tgt_all = tgt_ref[...]
        lane = lax.broadcasted_iota(jnp.int32, (B, 128), 1)
        tail = tail_ref[...]
        t_tail = jnp.sum(jnp.where(LAST_TILE + lane == tgt_all, tail, 0.0))
        sc_tail = jnp.sum(jnp.where(
            (tgt_all[:R_SC] != 0) & (lane[:R_SC] < V - LAST_TILE),
            tail[:R_SC], 0.0))
        acc_ref[0] = sc_tail
        acc_ref[1] = t_tail
        acc_ref[2] = jnp.sum((tgt_all != 0).astype(jnp.float32))

    tgt = tgt_ref[pl.ds(R_SC + pid * BR, BR), :]
    valid = tgt != 0
    blk = x_ref[...]
    rows = jnp.sum(blk, axis=1, keepdims=True)
    rows = rows - blk[:, 0:1]
    acc_ref[0] += jnp.sum(jnp.where(valid, rows, 0.0))

    @pl.when(pid == GR - 1)
    def _():
        part_ref[0, 0] = acc_ref[0]
        part_ref[0, 1] = acc_ref[1]
        part_ref[0, 2] = acc_ref[2]


def _combine_body(part_ref, t8_ref, tp_ref, p_ref, loss_ref):
    s_sc = 0.0
    p = p_ref[...]
    for q in range(8):
        vw = t8_ref[:, q:q + 1] != 0
        s_sc += jnp.sum(jnp.where(vw, p[:, q * 16:(q + 1) * 16], 0.0))
    s = part_ref[0, 0] + s_sc
    t = part_ref[0, 1] + jnp.sum(tp_ref[...])
    loss_ref[0, 0] = part_ref[0, 2] * C1 - SV * s - (CONF - SV) * t


def kernel(output, target):
    tpart, rpart = _build_sc_part()(output, target)
    part = pl.pallas_call(
        _tc_body,
        grid=(GR,),
        in_specs=[
            pl.BlockSpec((BR, V), lambda i: (SC_BLOCKS + i, 0)),
            pl.BlockSpec((B, 1), lambda i: (0, 0)),
            pl.BlockSpec((B, 128), lambda i: (0, LAST_TILE // 128)),
        ],
        out_specs=pl.BlockSpec((1, 4), lambda i: (0, 0),
                               memory_space=pltpu.SMEM),
        out_shape=jax.ShapeDtypeStruct((1, 4), jnp.float32),
        scratch_shapes=[pltpu.SMEM((3,), jnp.float32)],
    )(output, target.reshape(B, 1), output)
    loss = pl.pallas_call(
        _combine_body,
        in_specs=[
            pl.BlockSpec(memory_space=pltpu.SMEM),
            pl.BlockSpec((R_SC // 8, 8), lambda: (0, 0)),
            pl.BlockSpec((4, 128), lambda: (0, 0)),
            pl.BlockSpec((R_SC // 8, 128), lambda: (0, 0)),
        ],
        out_specs=pl.BlockSpec(memory_space=pltpu.SMEM),
        out_shape=jax.ShapeDtypeStruct((1, 1), jnp.float32),
    )(part, target[:R_SC].reshape(R_SC // 8, 8),
      tpart.reshape(4, 128), rpart.reshape(R_SC // 8, 128))
    return loss[0, 0]


# speedup vs baseline: 1.3650x; 1.3650x over previous
import functools
import math

import jax
import jax.numpy as jnp
from jax import lax
from jax.experimental import pallas as pl
from jax.experimental.pallas import tpu as pltpu
from jax.experimental.pallas import tpu_sc as plsc

V = 100000
B = 2048
SV = 0.1 / (V - 2)
CONF = 1.0 - 0.1
C1 = SV * math.log(SV) * (V - 2) + CONF * math.log(CONF)

NW = 32
PER_W = B // NW
LAST_TILE = (V // 128) * 128

R_SC = 512
ROWS_PW = R_SC // NW
BANDS_PW = ROWS_PW // 8
CTILES = 11
CW = CTILES * 128
NCH = LAST_TILE // CW

BR = 32
GR = (B - R_SC) // BR
SC_BLOCKS = R_SC // BR


@functools.lru_cache(maxsize=1)
def _build_sc_part():
    mesh = plsc.VectorSubcoreMesh(core_axis_name="c", subcore_axis_name="s")

    @functools.partial(
        pl.kernel,
        out_type=[
            jax.ShapeDtypeStruct((NW, 16), jnp.float32),
            jax.ShapeDtypeStruct((R_SC, 16), jnp.float32),
        ],
        mesh=mesh,
        scratch_types=[
            pltpu.VMEM((PER_W,), jnp.int32),
            pltpu.VMEM((PER_W, 8, 128), jnp.float32),
            pltpu.VMEM((16,), jnp.float32),
            pltpu.VMEM((2, 8, CW), jnp.float32),
            pltpu.VMEM((8, 16), jnp.float32),
            pltpu.SemaphoreType.DMA,
            pltpu.SemaphoreType.DMA,
        ],
    )
    def _sc_part(x_hbm, tgt_hbm, gout_hbm, rout_hbm,
                 tgt_v, tile_v, acc_v, buf_v, stage_v, sem_g, sem_r):
        wid = lax.axis_index("s") * 2 + lax.axis_index("c")
        lane = lax.broadcasted_iota(jnp.int32, (16,), 0)

        gbase = wid * PER_W
        pltpu.sync_copy(tgt_hbm.at[pl.ds(gbase, PER_W)], tgt_v)
        copies = []
        for j in range(PER_W):
            t = tgt_v[pl.ds((j // 16) * 16, 16)][j % 16]
            col = pl.multiple_of(
                jnp.minimum((t >> 7) << 7, LAST_TILE - 128), 128)
            row = pl.multiple_of((gbase + j) & ~7, 8)
            copies.append(pltpu.async_copy(
                x_hbm.at[pl.ds(row, 8), pl.ds(col, 128)],
                tile_v.at[j], sem_g))
        for c in copies:
            c.wait()
        acc = jnp.zeros((16,), jnp.float32)
        for j in range(PER_W):
            t = tgt_v[pl.ds((j // 16) * 16, 16)][j % 16]
            cs = pl.multiple_of(((t >> 4) & 7) * 16, 16)
            chunk = tile_v[j, j % 8, pl.ds(cs, 16)]
            ok = (t != 0) & (t < LAST_TILE)
            tt = jnp.where(ok, t & 15, -1)
            acc = acc + jnp.where(lane == tt, chunk, 0.0)
        acc_v[...] = acc
        pltpu.sync_copy(acc_v, gout_hbm.at[wid])

        rbase = wid * ROWS_PW

        def chunk_sum(b, accs, is0f):
            new = []
            for r in range(8):
                s = [buf_v[b, r, pl.ds(k * 16, 16)] for k in range(4)]
                for k in range(4, CW // 16):
                    s[k % 4] = s[k % 4] + buf_v[b, r, pl.ds(k * 16, 16)]
                a = accs[r] + ((s[0] + s[1]) + (s[2] + s[3]))
                a = a - is0f * jnp.where(
                    lane == 0, buf_v[b, r, pl.ds(0, 16)], 0.0)
                new.append(a)
            return tuple(new)

        def band_body(band, carry):
            row0 = pl.multiple_of(rbase + band * 8, 8)

            def issue(c, b):
                ccol = pl.multiple_of(c * CW, 128)
                pltpu.async_copy(
                    x_hbm.at[pl.ds(row0, 8), pl.ds(ccol, CW)],
                    buf_v.at[b], sem_r)

            def wait(b):
                pltpu.make_async_copy(
                    x_hbm.at[pl.ds(0, 8), pl.ds(0, CW)],
                    buf_v.at[b], sem_r).wait()

            issue(0, 0)

            def pair_body(i, accs):
                c = i * 2
                wait(0)
                issue(c + 1, 1)
                accs = chunk_sum(0, accs, jnp.where(i == 0, 1.0, 0.0))
                wait(1)
                issue(c + 2, 0)
                accs = chunk_sum(1, accs, 0.0)
                return accs

            zero = jnp.zeros((16,), jnp.float32)
            accs = lax.fori_loop(0, NCH // 2, pair_body, (zero,) * 8)
            wait(0)
            accs = chunk_sum(0, accs, 0.0)
            for r in range(8):
                stage_v[r, :] = accs[r]
            pltpu.sync_copy(stage_v, rout_hbm.at[pl.ds(row0, 8)])
            return carry

        lax.fori_loop(0, BANDS_PW, band_body, 0)

    return _sc_part


def _tc_body(x_ref, tgt_ref, tail_ref, part_ref, acc_ref):
    pid = pl.program_id(0)

    @pl.when(pid == 0)
    def _():
        tgt_all = tgt_ref[...]
        lane = lax.broadcasted_iota(jnp.int32, (B, 128), 1)
        tail = tail_ref[...]
        t_tail = jnp.sum(jnp.where(LAST_TILE + lane == tgt_all, tail, 0.0))
        sc_tail = jnp.sum(jnp.where(
            (tgt_all[:R_SC] != 0) & (lane[:R_SC] < V - LAST_TILE),
            tail[:R_SC], 0.0))
        acc_ref[0] = sc_tail
        acc_ref[1] = t_tail
        acc_ref[2] = jnp.sum((tgt_all != 0).astype(jnp.float32))

    tgt = tgt_ref[pl.ds(R_SC + pid * BR, BR), :]
    valid = tgt != 0
    blk = x_ref[...]
    rows = jnp.sum(blk, axis=1, keepdims=True)
    rows = rows - blk[:, 0:1]
    acc_ref[0] += jnp.sum(jnp.where(valid, rows, 0.0))

    @pl.when(pid == GR - 1)
    def _():
        part_ref[0, 0] = acc_ref[0]
        part_ref[0, 1] = acc_ref[1]
        part_ref[0, 2] = acc_ref[2]


def _combine_body(part_ref, t8_ref, tp_ref, p_ref, loss_ref):
    s_sc = 0.0
    p = p_ref[...]
    for q in range(8):
        vw = t8_ref[:, q:q + 1] != 0
        s_sc += jnp.sum(jnp.where(vw, p[:, q * 16:(q + 1) * 16], 0.0))
    s = part_ref[0, 0] + s_sc
    t = part_ref[0, 1] + jnp.sum(tp_ref[...])
    loss_ref[0, 0] = part_ref[0, 2] * C1 - SV * s - (CONF - SV) * t


def kernel(output, target):
    tpart, rpart = _build_sc_part()(output, target)
    part = pl.pallas_call(
        _tc_body,
        grid=(GR,),
        in_specs=[
            pl.BlockSpec((BR, V), lambda i: (SC_BLOCKS + i, 0)),
            pl.BlockSpec((B, 1), lambda i: (0, 0)),
            pl.BlockSpec((B, 128), lambda i: (0, LAST_TILE // 128)),
        ],
        out_specs=pl.BlockSpec((1, 4), lambda i: (0, 0),
                               memory_space=pltpu.SMEM),
        out_shape=jax.ShapeDtypeStruct((1, 4), jnp.float32),
        scratch_shapes=[pltpu.SMEM((3,), jnp.float32)],
    )(output, target.reshape(B, 1), output)
    loss = pl.pallas_call(
        _combine_body,
        in_specs=[
            pl.BlockSpec(memory_space=pltpu.SMEM),
            pl.BlockSpec((R_SC // 8, 8), lambda: (0, 0)),
            pl.BlockSpec((4, 128), lambda: (0, 0)),
            pl.BlockSpec((R_SC // 8, 128), lambda: (0, 0)),
        ],
        out_specs=pl.BlockSpec(memory_space=pltpu.SMEM),
        out_shape=jax.ShapeDtypeStruct((1, 1), jnp.float32),
    )(part, target[:R_SC].reshape(R_SC // 8, 8),
      tpart.reshape(4, 128), rpart.reshape(R_SC // 8, 128))
    return loss[0, 0]


# speedup vs baseline: 1.5302x; 1.1210x over previous
import functools
import math

import jax
import jax.numpy as jnp
from jax import lax
from jax.experimental import pallas as pl
from jax.experimental.pallas import tpu as pltpu
from jax.experimental.pallas import tpu_sc as plsc

V = 100000
B = 2048
SV = 0.1 / (V - 2)
CONF = 1.0 - 0.1
C1 = SV * math.log(SV) * (V - 2) + CONF * math.log(CONF)

BC = 1024
NB = (V + BC - 1) // BC
G = NB // 2
TAIL = V - (NB - 1) * BC
TAIL_FULL = TAIL // 128
TAIL_REM = TAIL % 128

NW = 32
PER_W = B // NW
LAST_TILE = (V // 128) * 128


@functools.lru_cache(maxsize=1)
def _build_sc_gather():
    mesh = plsc.VectorSubcoreMesh(core_axis_name="c", subcore_axis_name="s")

    @functools.partial(
        pl.kernel,
        out_type=jax.ShapeDtypeStruct((NW, 16), jnp.float32),
        mesh=mesh,
        scratch_types=[
            pltpu.VMEM((PER_W,), jnp.int32),
            pltpu.VMEM((PER_W, 8, 128), jnp.float32),
            pltpu.VMEM((16,), jnp.float32),
            pltpu.SemaphoreType.DMA,
        ],
    )
    def _sc_gather(x_hbm, tgt_hbm, out_hbm, tgt_v, tile_v, acc_v, sem):
        wid = lax.axis_index("s") * 2 + lax.axis_index("c")
        base = wid * PER_W
        pltpu.sync_copy(tgt_hbm.at[pl.ds(base, PER_W)], tgt_v)
        copies = []
        for j in range(PER_W):
            t = tgt_v[pl.ds((j // 16) * 16, 16)][j % 16]
            col = pl.multiple_of(
                jnp.minimum((t >> 7) << 7, LAST_TILE - 128), 128)
            row = pl.multiple_of((base + j) & ~7, 8)
            copies.append(pltpu.async_copy(
                x_hbm.at[pl.ds(row, 8), pl.ds(col, 128)],
                tile_v.at[j], sem))
        for c in copies:
            c.wait()
        acc = jnp.zeros((16,), jnp.float32)
        lane = lax.broadcasted_iota(jnp.int32, (16,), 0)
        for j in range(PER_W):
            t = tgt_v[pl.ds((j // 16) * 16, 16)][j % 16]
            cs = pl.multiple_of(((t >> 4) & 7) * 16, 16)
            chunk = tile_v[j, j % 8, pl.ds(cs, 16)]
            ok = (t != 0) & (t < LAST_TILE)
            tt = jnp.where(ok, t & 15, -1)
            acc = acc + jnp.where(lane == tt, chunk, 0.0)
        acc_v[...] = acc
        pltpu.sync_copy(acc_v, out_hbm.at[wid])

    return _sc_gather


BR = 64
GR = B // BR


def _tc_body(x_ref, tgt_ref, tp_ref, loss_ref, acc_ref):
    pid = pl.program_id(0)

    @pl.when(pid == 0)
    def _():
        acc_ref[0] = 0.0
        acc_ref[1] = 0.0
        acc_ref[2] = 0.0

    tgt = tgt_ref[...]
    valid = tgt != 0
    blk = x_ref[...]
    rows = jnp.sum(blk, axis=1, keepdims=True)
    rows = rows - blk[:, 0:1]
    acc_ref[0] += jnp.sum(jnp.where(valid, rows, 0.0))
    tail = blk[:, LAST_TILE:V]
    lane = lax.broadcasted_iota(jnp.int32, (BR, V - LAST_TILE), 1)
    acc_ref[1] += jnp.sum(jnp.where(LAST_TILE + lane == tgt, tail, 0.0))
    acc_ref[2] += jnp.sum(valid.astype(jnp.float32))

    @pl.when(pid == GR - 1)
    def _():
        t = jnp.sum(tp_ref[...]) + acc_ref[1]
        loss_ref[0, 0] = acc_ref[2] * C1 - SV * acc_ref[0] - (CONF - SV) * t


def kernel(output, target):
    tpart = _build_sc_gather()(output, target)
    loss = pl.pallas_call(
        _tc_body,
        grid=(GR,),
        in_specs=[
            pl.BlockSpec((BR, V), lambda i: (i, 0)),
            pl.BlockSpec((BR, 1), lambda i: (i, 0)),
            pl.BlockSpec((4, 128), lambda i: (0, 0)),
        ],
        out_specs=pl.BlockSpec((1, 1), lambda i: (0, 0),
                               memory_space=pltpu.SMEM),
        out_shape=jax.ShapeDtypeStruct((1, 1), jnp.float32),
        scratch_shapes=[pltpu.SMEM((3,), jnp.float32)],
    )(output, target.reshape(B, 1), tpart.reshape(4, 128))
    return loss[0, 0]
